# baseline (speedup 1.0000x reference)
import functools

import jax
import jax.numpy as jnp
from jax import lax
from jax.experimental import pallas as pl
from jax.experimental.pallas import tpu as pltpu
from jax.experimental.pallas import tpu_sc as plsc

MASK_RATIO = 0.75

NC = 2
NS = 16
NW = NC * NS


def _sc_random_mask(x2, gidx, pos, rvals, mvals, n_keep_rows, n_total):
    dim = x2.shape[1]
    g_per_w = n_keep_rows // NW
    s_per_w = n_total // NW
    GC = 64
    SC_CH = 128
    n_gc = g_per_w // GC
    n_sc = s_per_w // SC_CH

    mesh = plsc.VectorSubcoreMesh(core_axis_name="c", subcore_axis_name="s")

    @functools.partial(
        pl.kernel,
        mesh=mesh,
        out_type=[
            jax.ShapeDtypeStruct((n_keep_rows, dim), jnp.float32),
            jax.ShapeDtypeStruct((n_total,), jnp.int32),
            jax.ShapeDtypeStruct((n_total,), jnp.float32),
        ],
        scratch_types=[
            pltpu.VMEM((GC,), jnp.int32),
            pltpu.VMEM((GC, dim), jnp.float32),
            pltpu.VMEM((GC, dim), jnp.float32),
            pltpu.VMEM((SC_CH,), jnp.int32),
            pltpu.VMEM((SC_CH,), jnp.int32),
            pltpu.VMEM((SC_CH,), jnp.float32),
            pltpu.SemaphoreType.DMA,
            pltpu.SemaphoreType.DMA,
        ],
    )
    def body(x_hbm, gidx_hbm, pos_hbm, rv_hbm, mv_hbm,
             xe_hbm, restore_hbm, mask_hbm,
             idx_v, rows_a, rows_b, pos_v, rv_v, mv_v, sem_g, sem_s):
        wid = lax.axis_index("s") * NC + lax.axis_index("c")
        gbase = wid * g_per_w
        sbase = wid * s_per_w

        for c in range(n_sc):
            off = sbase + c * SC_CH
            pltpu.sync_copy(pos_hbm.at[pl.ds(off, SC_CH)], pos_v)
            pltpu.sync_copy(rv_hbm.at[pl.ds(off, SC_CH)], rv_v)
            pltpu.sync_copy(mv_hbm.at[pl.ds(off, SC_CH)], mv_v)
            cp1 = pltpu.async_copy(rv_v, restore_hbm.at[pos_v], sem_s)
            cp2 = pltpu.async_copy(mv_v, mask_hbm.at[pos_v], sem_s)
            cp1.wait()
            cp2.wait()

        bufs = (rows_a, rows_b)
        cps = [None, None]
        for c in range(n_gc + 1):
            if c < n_gc:
                pltpu.sync_copy(gidx_hbm.at[pl.ds(gbase + c * GC, GC)], idx_v)
                cps[c % 2] = pltpu.async_copy(x_hbm.at[idx_v], bufs[c % 2], sem_g)
            if c >= 1:
                cps[(c - 1) % 2].wait()
                pltpu.sync_copy(bufs[(c - 1) % 2],
                                xe_hbm.at[pl.ds(gbase + (c - 1) * GC, GC)])

    return body(x2, gidx, pos, rvals, mvals)


def kernel(x):
    batch, length, dim = x.shape
    num_keep = int(length * (1 - MASK_RATIO))
    n_total = batch * length

    noise = jax.random.uniform(jax.random.key(42), (batch, length),
                               dtype=jnp.float32)
    ids_shuffle = jnp.argsort(noise, axis=1).astype(jnp.int32)

    boff = (jnp.arange(batch, dtype=jnp.int32) * length)[:, None]
    pos = (ids_shuffle + boff).reshape(-1)
    gidx = (ids_shuffle[:, :num_keep] + boff).reshape(-1)
    rank = jnp.tile(jnp.arange(length, dtype=jnp.int32), batch)
    rvals = rank
    mvals = jnp.where(rank < num_keep, 0.0, 1.0).astype(jnp.float32)

    x2 = x.reshape(n_total, dim)
    xe_flat, restore_flat, mask_flat = _sc_random_mask(
        x2, gidx, pos, rvals, mvals, batch * num_keep, n_total)

    return (xe_flat.reshape(batch, num_keep, dim),
            mask_flat.reshape(batch, length),
            restore_flat.reshape(batch, length))
